# baseline (speedup 1.0000x reference)
import functools
import math

import jax
import jax.numpy as jnp
from jax import lax
from jax.experimental import pallas as pl
from jax.experimental.pallas import tpu as pltpu
from jax.experimental.pallas import tpu_sc as plsc

B, S, D = 16, 512, 512
HV = 50002
NF = 3
GROUP = 4
FF = 1376
NL = 2
PATCH = 6
PRIMES = (31, 53, 97)
MULTS = tuple(tuple(pow(p, i, HV) for i in range(GROUP)) for p in PRIMES)

_NC, _NS = 2, 16
_NW = _NC * _NS
_TOK_PER_W = (B * S) // _NW
_CHUNK = 64


def _sc_embeds_body(ids_hbm, tok_hbm, hash_hbm, out_hbm, ids_v, hidx_v, acc_v,
                    buf_v, sem):
    cid = lax.axis_index("c")
    sid = lax.axis_index("s")
    wid = cid * _NS + sid
    base = wid * _TOK_PER_W

    ids_v[pl.ds(0, 16)] = jnp.zeros((16,), jnp.int32)
    pltpu.sync_copy(ids_hbm.at[pl.ds(base, _TOK_PER_W)], ids_v.at[pl.ds(8, _TOK_PER_W)])

    @pl.when(wid % 2 == 1)
    def _():
        pltpu.sync_copy(ids_hbm.at[pl.ds(base - 8, 8)], ids_v.at[pl.ds(0, 8)])

    for c in range(_TOK_PER_W // _CHUNK):
        for v in range(_CHUNK // 16):
            t0 = 8 + c * _CHUNK + v * 16
            w0 = ids_v[pl.ds(t0, 16)]
            w1 = ids_v[pl.ds(t0 - 1, 16)]
            w2 = ids_v[pl.ds(t0 - 2, 16)]
            w3 = ids_v[pl.ds(t0 - 3, 16)]
            for f in range(NF):
                m = MULTS[f]
                h = (w0 * m[0] + w1 * m[1] + w2 * m[2] + w3 * m[3]) % HV + f * HV
                hidx_v[pl.ds(f * _CHUNK + v * 16, 16)] = h
        pltpu.async_copy(tok_hbm.at[ids_v.at[pl.ds(8 + c * _CHUNK, _CHUNK)]],
                         acc_v, sem).wait()
        for f in range(NF):
            pltpu.async_copy(hash_hbm.at[hidx_v.at[pl.ds(f * _CHUNK, _CHUNK)]],
                             buf_v, sem).wait()

            @plsc.parallel_loop(0, _CHUNK)
            def _(r):
                for j in range(D // 16):
                    plsc.addupdate(acc_v.at[r, pl.ds(j * 16, 16)],
                                   buf_v[r, pl.ds(j * 16, 16)])
        pltpu.sync_copy(acc_v, out_hbm.at[pl.ds(base + c * _CHUNK, _CHUNK)])


def _sc_embeds(ids_flat, tok_emb, hash_flat):
    mesh = plsc.VectorSubcoreMesh(core_axis_name="c", subcore_axis_name="s")
    fn = functools.partial(
        pl.kernel,
        out_type=jax.ShapeDtypeStruct((B * S, D), jnp.float32),
        mesh=mesh,
        scratch_types=[
            pltpu.VMEM((264,), jnp.int32),
            pltpu.VMEM((NF * _CHUNK,), jnp.int32),
            pltpu.VMEM((_CHUNK, D), jnp.float32),
            pltpu.VMEM((_CHUNK, D), jnp.float32),
            pltpu.SemaphoreType.DMA,
        ],
    )(_sc_embeds_body)
    return fn(ids_flat, tok_emb, hash_flat)


def _rms(x, g):
    return x * lax.rsqrt(jnp.mean(x * x, axis=-1, keepdims=True) + 1e-5) * g


def _rope(x, cos, sin):
    x1 = x[:, : D // 2]
    x2 = x[:, D // 2:]
    return jnp.concatenate([x1 * cos - x2 * sin, x1 * sin + x2 * cos], axis=-1)


def _bdot(a, b):
    return jnp.dot(a.astype(jnp.bfloat16), b.astype(jnp.bfloat16),
                   preferred_element_type=jnp.float32)


def _layer(x, wq, wk, wv, wo, w1, w3, w2, g0, g1, cos, sin):
    h = _rms(x, g0)
    q = _rope(_bdot(h, wq), cos, sin)
    k = _rope(_bdot(h, wk), cos, sin)
    v = _bdot(h, wv)
    s = lax.dot_general(q.astype(jnp.bfloat16), k.astype(jnp.bfloat16),
                        (((1,), (1,)), ((), ())),
                        preferred_element_type=jnp.float32)
    s = s * (1.0 / math.sqrt(float(D)))
    rows = lax.broadcasted_iota(jnp.int32, (S, S), 0)
    cols = lax.broadcasted_iota(jnp.int32, (S, S), 1)
    s = jnp.where(rows >= cols, s, jnp.float32(-1e9))
    s = s - jnp.max(s, axis=-1, keepdims=True)
    e = jnp.exp(s)
    attn = e / jnp.sum(e, axis=-1, keepdims=True)
    av = _bdot(attn, v)
    x = x + _bdot(av, wo)
    h = _rms(x, g1)
    ff = jax.nn.silu(_bdot(h, w1))
    ff = ff * _bdot(h, w3)
    return x + _bdot(ff, w2)


def _pool_gather(x):
    t = lax.broadcasted_iota(jnp.int32, (S, 1), 0)
    g = x
    ninf = jnp.float32(-jnp.inf)
    for d in range(1, PATCH):
        up = jnp.concatenate([x[d:], x[:d]], axis=0)
        cu = (t + d < S) & ((t + d) // PATCH == t // PATCH)
        g = jnp.maximum(g, jnp.where(cu, up, ninf))
        dn = jnp.concatenate([x[S - d:], x[:S - d]], axis=0)
        cd = (t - d >= 0) & ((t - d) // PATCH == t // PATCH)
        g = jnp.maximum(g, jnp.where(cd, dn, ninf))
    return g


def _enc_body(emb_ref, aw_ref, w1_ref, w3_ref, w2_ref, nrm_ref, cos_ref, sin_ref,
              out_ref):
    cos = cos_ref[...]
    sin = sin_ref[...]
    e = emb_ref[0]
    x = e
    for l in range(NL):
        x = _layer(x, aw_ref[l, 0], aw_ref[l, 1], aw_ref[l, 2], aw_ref[l, 3],
                   w1_ref[l], w3_ref[l], w2_ref[l], nrm_ref[l, 0], nrm_ref[l, 1],
                   cos, sin)
    out_ref[0] = e + _pool_gather(x)


def _dec_body(y_ref, aw_ref, w1_ref, w3_ref, w2_ref, nrm_ref, fn_ref, cos_ref,
              sin_ref, out_ref):
    cos = cos_ref[...]
    sin = sin_ref[...]
    x = y_ref[0]
    for l in range(NL):
        x = _layer(x, aw_ref[l, 0], aw_ref[l, 1], aw_ref[l, 2], aw_ref[l, 3],
                   w1_ref[l], w3_ref[l], w2_ref[l], nrm_ref[l, 0], nrm_ref[l, 1],
                   cos, sin)
    out_ref[0] = _rms(x, fn_ref[0])


def _full_spec(shape):
    n = len(shape)
    return pl.BlockSpec(shape, lambda b: (0,) * n)


def _tc_call(body, n_extra_specs, extra_args, x, aw, w1, w3, w2, nrm, cos, sin):
    in_specs = [
        pl.BlockSpec((1, S, D), lambda b: (b, 0, 0)),
        _full_spec(aw.shape),
        _full_spec(w1.shape),
        _full_spec(w3.shape),
        _full_spec(w2.shape),
        _full_spec(nrm.shape),
    ] + n_extra_specs + [
        _full_spec(cos.shape),
        _full_spec(sin.shape),
    ]
    return pl.pallas_call(
        body,
        grid=(B,),
        in_specs=in_specs,
        out_specs=pl.BlockSpec((1, S, D), lambda b: (b, 0, 0)),
        out_shape=jax.ShapeDtypeStruct((B, S, D), jnp.float32),
        compiler_params=pltpu.CompilerParams(
            dimension_semantics=("arbitrary",),
            vmem_limit_bytes=100 * 1024 * 1024,
        ),
    )(x, aw, w1, w3, w2, nrm, *extra_args, cos, sin)


def kernel(input_ids, tok_emb, hash_emb, enc_attn_w, enc_w1, enc_w3, enc_w2,
           enc_norms, dec_attn_w, dec_w1, dec_w3, dec_w2, dec_norms, final_norm):
    ids_flat = input_ids.astype(jnp.int32).reshape(B * S)
    hash_flat = hash_emb.reshape(NF * HV, D)

    embeds = _sc_embeds(ids_flat, tok_emb, hash_flat).reshape(B, S, D)

    pos = jnp.arange(S, dtype=jnp.float32)
    freqs = 1.0 / (10000.0 ** (jnp.arange(D // 2, dtype=jnp.float32) / (D // 2)))
    ang = pos[:, None] * freqs[None, :]
    cos, sin = jnp.cos(ang), jnp.sin(ang)

    y0 = _tc_call(_enc_body, [], [], embeds, enc_attn_w, enc_w1, enc_w3, enc_w2,
                  enc_norms, cos, sin)

    fn2 = final_norm.reshape(1, D)
    out = _tc_call(_dec_body, [_full_spec(fn2.shape)], [fn2], y0, dec_attn_w,
                   dec_w1, dec_w3, dec_w2, dec_norms, cos, sin)
    return out


# speedup vs baseline: 1.4760x; 1.4760x over previous
import functools
import math

import jax
import jax.numpy as jnp
from jax import lax
from jax.experimental import pallas as pl
from jax.experimental.pallas import tpu as pltpu
from jax.experimental.pallas import tpu_sc as plsc

B, S, D = 16, 512, 512
HV = 50002
NF = 3
GROUP = 4
FF = 1376
NL = 2
PATCH = 6
PRIMES = (31, 53, 97)
MULTS = tuple(tuple(pow(p, i, HV) for i in range(GROUP)) for p in PRIMES)

_NC, _NS = 2, 16
_NW = _NC * _NS
_TOK_PER_W = (B * S) // _NW
_CHUNK = 64


def _sc_embeds_body(ids_hbm, tok_hbm, hash_hbm, out_hbm, ids_v, hidx_v, acc_v,
                    buf_v, sem):
    cid = lax.axis_index("c")
    sid = lax.axis_index("s")
    wid = cid * _NS + sid
    base = wid * _TOK_PER_W

    ids_v[pl.ds(0, 16)] = jnp.zeros((16,), jnp.int32)
    pltpu.sync_copy(ids_hbm.at[pl.ds(base, _TOK_PER_W)], ids_v.at[pl.ds(8, _TOK_PER_W)])

    @pl.when(wid % 2 == 1)
    def _():
        pltpu.sync_copy(ids_hbm.at[pl.ds(base - 8, 8)], ids_v.at[pl.ds(0, 8)])

    for c in range(_TOK_PER_W // _CHUNK):
        for v in range(_CHUNK // 16):
            t0 = 8 + c * _CHUNK + v * 16
            w0 = ids_v[pl.ds(t0, 16)]
            w1 = ids_v[pl.ds(t0 - 1, 16)]
            w2 = ids_v[pl.ds(t0 - 2, 16)]
            w3 = ids_v[pl.ds(t0 - 3, 16)]
            for f in range(NF):
                m = MULTS[f]
                h = (w0 * m[0] + w1 * m[1] + w2 * m[2] + w3 * m[3]) % HV
                hidx_v[pl.ds(f * _CHUNK + v * 16, 16)] = h
        pltpu.async_copy(tok_hbm.at[ids_v.at[pl.ds(8 + c * _CHUNK, _CHUNK)]],
                         acc_v, sem).wait()
        for f in range(NF):
            pltpu.async_copy(hash_hbm.at[f].at[hidx_v.at[pl.ds(f * _CHUNK, _CHUNK)]],
                             buf_v, sem).wait()

            @plsc.parallel_loop(0, _CHUNK)
            def _(r):
                for j in range(D // 16):
                    plsc.addupdate(acc_v.at[r, pl.ds(j * 16, 16)],
                                   buf_v[r, pl.ds(j * 16, 16)])
        pltpu.sync_copy(acc_v, out_hbm.at[pl.ds(base + c * _CHUNK, _CHUNK)])


def _sc_embeds(ids_flat, tok_emb, hash_flat):
    mesh = plsc.VectorSubcoreMesh(core_axis_name="c", subcore_axis_name="s")
    fn = functools.partial(
        pl.kernel,
        out_type=jax.ShapeDtypeStruct((B * S, D), jnp.float32),
        mesh=mesh,
        scratch_types=[
            pltpu.VMEM((264,), jnp.int32),
            pltpu.VMEM((NF * _CHUNK,), jnp.int32),
            pltpu.VMEM((_CHUNK, D), jnp.float32),
            pltpu.VMEM((_CHUNK, D), jnp.float32),
            pltpu.SemaphoreType.DMA,
        ],
    )(_sc_embeds_body)
    return fn(ids_flat, tok_emb, hash_flat)


def _rms(x, g):
    return x * lax.rsqrt(jnp.mean(x * x, axis=-1, keepdims=True) + 1e-5) * g


def _rope(x, cos, sin):
    x1 = x[:, : D // 2]
    x2 = x[:, D // 2:]
    return jnp.concatenate([x1 * cos - x2 * sin, x1 * sin + x2 * cos], axis=-1)


def _bdot(a, b):
    return jnp.dot(a.astype(jnp.bfloat16), b.astype(jnp.bfloat16),
                   preferred_element_type=jnp.float32)


def _layer(x, wq, wk, wv, wo, w1, w3, w2, g0, g1, cos, sin):
    h = _rms(x, g0)
    q = _rope(_bdot(h, wq), cos, sin)
    k = _rope(_bdot(h, wk), cos, sin)
    v = _bdot(h, wv)
    s = lax.dot_general(q.astype(jnp.bfloat16), k.astype(jnp.bfloat16),
                        (((1,), (1,)), ((), ())),
                        preferred_element_type=jnp.float32)
    s = s * (1.0 / math.sqrt(float(D)))
    rows = lax.broadcasted_iota(jnp.int32, (S, S), 0)
    cols = lax.broadcasted_iota(jnp.int32, (S, S), 1)
    s = jnp.where(rows >= cols, s, jnp.float32(-1e9))
    s = s - jnp.max(s, axis=-1, keepdims=True)
    e = jnp.exp(s)
    attn = e / jnp.sum(e, axis=-1, keepdims=True)
    av = _bdot(attn, v)
    x = x + _bdot(av, wo)
    h = _rms(x, g1)
    ff = jax.nn.silu(_bdot(h, w1))
    ff = ff * _bdot(h, w3)
    return x + _bdot(ff, w2)


def _pool_gather(x):
    t = lax.broadcasted_iota(jnp.int32, (S, 1), 0)
    g = x
    ninf = jnp.float32(-jnp.inf)
    for d in range(1, PATCH):
        up = jnp.concatenate([x[d:], x[:d]], axis=0)
        cu = (t + d < S) & ((t + d) // PATCH == t // PATCH)
        g = jnp.maximum(g, jnp.where(cu, up, ninf))
        dn = jnp.concatenate([x[S - d:], x[:S - d]], axis=0)
        cd = (t - d >= 0) & ((t - d) // PATCH == t // PATCH)
        g = jnp.maximum(g, jnp.where(cd, dn, ninf))
    return g


def _enc_body(emb_ref, aw_ref, w1_ref, w3_ref, w2_ref, nrm_ref, cos_ref, sin_ref,
              out_ref):
    cos = cos_ref[...]
    sin = sin_ref[...]
    e = emb_ref[0]
    x = e
    for l in range(NL):
        x = _layer(x, aw_ref[l, 0], aw_ref[l, 1], aw_ref[l, 2], aw_ref[l, 3],
                   w1_ref[l], w3_ref[l], w2_ref[l], nrm_ref[l, 0], nrm_ref[l, 1],
                   cos, sin)
    out_ref[0] = e + _pool_gather(x)


def _dec_body(y_ref, aw_ref, w1_ref, w3_ref, w2_ref, nrm_ref, fn_ref, cos_ref,
              sin_ref, out_ref):
    cos = cos_ref[...]
    sin = sin_ref[...]
    x = y_ref[0]
    for l in range(NL):
        x = _layer(x, aw_ref[l, 0], aw_ref[l, 1], aw_ref[l, 2], aw_ref[l, 3],
                   w1_ref[l], w3_ref[l], w2_ref[l], nrm_ref[l, 0], nrm_ref[l, 1],
                   cos, sin)
    out_ref[0] = _rms(x, fn_ref[0])


def _full_spec(shape):
    n = len(shape)
    return pl.BlockSpec(shape, lambda b: (0,) * n)


def _tc_call(body, n_extra_specs, extra_args, x, aw, w1, w3, w2, nrm, cos, sin):
    in_specs = [
        pl.BlockSpec((1, S, D), lambda b: (b, 0, 0)),
        _full_spec(aw.shape),
        _full_spec(w1.shape),
        _full_spec(w3.shape),
        _full_spec(w2.shape),
        _full_spec(nrm.shape),
    ] + n_extra_specs + [
        _full_spec(cos.shape),
        _full_spec(sin.shape),
    ]
    return pl.pallas_call(
        body,
        grid=(B,),
        in_specs=in_specs,
        out_specs=pl.BlockSpec((1, S, D), lambda b: (b, 0, 0)),
        out_shape=jax.ShapeDtypeStruct((B, S, D), jnp.float32),
        compiler_params=pltpu.CompilerParams(
            dimension_semantics=("arbitrary",),
            vmem_limit_bytes=100 * 1024 * 1024,
        ),
    )(x, aw, w1, w3, w2, nrm, *extra_args, cos, sin)


def kernel(input_ids, tok_emb, hash_emb, enc_attn_w, enc_w1, enc_w3, enc_w2,
           enc_norms, dec_attn_w, dec_w1, dec_w3, dec_w2, dec_norms, final_norm):
    ids_flat = input_ids.astype(jnp.int32).reshape(B * S)

    embeds = _sc_embeds(ids_flat, tok_emb, hash_emb).reshape(B, S, D)

    pos = jnp.arange(S, dtype=jnp.float32)
    freqs = 1.0 / (10000.0 ** (jnp.arange(D // 2, dtype=jnp.float32) / (D // 2)))
    ang = pos[:, None] * freqs[None, :]
    cos, sin = jnp.cos(ang), jnp.sin(ang)

    y0 = _tc_call(_enc_body, [], [], embeds, enc_attn_w, enc_w1, enc_w3, enc_w2,
                  enc_norms, cos, sin)

    fn2 = final_norm.reshape(1, D)
    out = _tc_call(_dec_body, [_full_spec(fn2.shape)], [fn2], y0, dec_attn_w,
                   dec_w1, dec_w3, dec_w2, dec_norms, cos, sin)
    return out


# speedup vs baseline: 1.5050x; 1.0197x over previous
import functools
import math

import jax
import jax.numpy as jnp
from jax import lax
from jax.experimental import pallas as pl
from jax.experimental.pallas import tpu as pltpu
from jax.experimental.pallas import tpu_sc as plsc

B, S, D = 16, 512, 512
HV = 50002
NF = 3
GROUP = 4
FF = 1376
NL = 2
PATCH = 6
PRIMES = (31, 53, 97)
MULTS = tuple(tuple(pow(p, i, HV) for i in range(GROUP)) for p in PRIMES)

_NC, _NS = 2, 16
_NW = _NC * _NS
_TOK_PER_W = (B * S) // _NW
_CHUNK = 64


def _sc_embeds_body(ids_hbm, tok_hbm, hash_hbm, out_hbm, ids_v, hidx_v, acc_v,
                    buf_v, sem):
    cid = lax.axis_index("c")
    sid = lax.axis_index("s")
    wid = cid * _NS + sid
    base = wid * _TOK_PER_W

    ids_v[pl.ds(0, 16)] = jnp.zeros((16,), jnp.int32)
    pltpu.sync_copy(ids_hbm.at[pl.ds(base, _TOK_PER_W)], ids_v.at[pl.ds(8, _TOK_PER_W)])

    @pl.when(wid % 2 == 1)
    def _():
        pltpu.sync_copy(ids_hbm.at[pl.ds(base - 8, 8)], ids_v.at[pl.ds(0, 8)])

    for c in range(_TOK_PER_W // _CHUNK):
        for v in range(_CHUNK // 16):
            t0 = 8 + c * _CHUNK + v * 16
            w0 = ids_v[pl.ds(t0, 16)]
            w1 = ids_v[pl.ds(t0 - 1, 16)]
            w2 = ids_v[pl.ds(t0 - 2, 16)]
            w3 = ids_v[pl.ds(t0 - 3, 16)]
            for f in range(NF):
                m = MULTS[f]
                h = (w0 * m[0] + w1 * m[1] + w2 * m[2] + w3 * m[3]) % HV
                hidx_v[pl.ds(f * _CHUNK + v * 16, 16)] = h
        pltpu.async_copy(tok_hbm.at[ids_v.at[pl.ds(8 + c * _CHUNK, _CHUNK)]],
                         acc_v, sem).wait()
        for f in range(NF):
            pltpu.async_copy(hash_hbm.at[f].at[hidx_v.at[pl.ds(f * _CHUNK, _CHUNK)]],
                             buf_v, sem).wait()

            @plsc.parallel_loop(0, _CHUNK)
            def _(r):
                for j in range(D // 16):
                    plsc.addupdate(acc_v.at[r, pl.ds(j * 16, 16)],
                                   buf_v[r, pl.ds(j * 16, 16)])
        pltpu.sync_copy(acc_v, out_hbm.at[pl.ds(base + c * _CHUNK, _CHUNK)])


def _sc_embeds(ids_flat, tok_emb, hash_flat):
    mesh = plsc.VectorSubcoreMesh(core_axis_name="c", subcore_axis_name="s")
    fn = functools.partial(
        pl.kernel,
        out_type=jax.ShapeDtypeStruct((B * S, D), jnp.float32),
        mesh=mesh,
        scratch_types=[
            pltpu.VMEM((264,), jnp.int32),
            pltpu.VMEM((NF * _CHUNK,), jnp.int32),
            pltpu.VMEM((_CHUNK, D), jnp.float32),
            pltpu.VMEM((_CHUNK, D), jnp.float32),
            pltpu.SemaphoreType.DMA,
        ],
    )(_sc_embeds_body)
    return fn(ids_flat, tok_emb, hash_flat)


def _rms(x, g):
    return x * lax.rsqrt(jnp.mean(x * x, axis=-1, keepdims=True) + 1e-5) * g


def _rope(x, cos, sin):
    x1 = x[:, : D // 2]
    x2 = x[:, D // 2:]
    return jnp.concatenate([x1 * cos - x2 * sin, x1 * sin + x2 * cos], axis=-1)


def _fdot(a16, b16):
    return jnp.dot(a16, b16, preferred_element_type=jnp.float32)


def _layer(x, wq, wk, wv, wo, w1, w3, w2, g0, g1, cos, sin):
    bf = jnp.bfloat16
    h16 = _rms(x, g0).astype(bf)
    q16 = _rope(_fdot(h16, wq), cos, sin).astype(bf)
    k16 = _rope(_fdot(h16, wk), cos, sin).astype(bf)
    v16 = _fdot(h16, wv).astype(bf)
    s = lax.dot_general(q16, k16, (((1,), (1,)), ((), ())),
                        preferred_element_type=jnp.float32)
    s = s * (1.0 / math.sqrt(float(D)))
    rows = lax.broadcasted_iota(jnp.int32, (S, S), 0)
    cols = lax.broadcasted_iota(jnp.int32, (S, S), 1)
    s = jnp.where(rows >= cols, s, jnp.float32(-1e9))
    s = s - jnp.max(s, axis=-1, keepdims=True)
    e = jnp.exp(s)
    attn16 = (e / jnp.sum(e, axis=-1, keepdims=True)).astype(bf)
    av16 = _fdot(attn16, v16).astype(bf)
    x = x + _fdot(av16, wo)
    h16 = _rms(x, g1).astype(bf)
    ff16 = (jax.nn.silu(_fdot(h16, w1)) * _fdot(h16, w3)).astype(bf)
    return x + _fdot(ff16, w2)


def _pool_gather(x):
    t = lax.broadcasted_iota(jnp.int32, (S, 1), 0)
    g = x
    ninf = jnp.float32(-jnp.inf)
    for d in range(1, PATCH):
        up = jnp.concatenate([x[d:], x[:d]], axis=0)
        cu = (t + d < S) & ((t + d) // PATCH == t // PATCH)
        g = jnp.maximum(g, jnp.where(cu, up, ninf))
        dn = jnp.concatenate([x[S - d:], x[:S - d]], axis=0)
        cd = (t - d >= 0) & ((t - d) // PATCH == t // PATCH)
        g = jnp.maximum(g, jnp.where(cd, dn, ninf))
    return g


def _enc_body(emb_ref, aw_ref, w1_ref, w3_ref, w2_ref, nrm_ref, cos_ref, sin_ref,
              out_ref):
    cos = cos_ref[...]
    sin = sin_ref[...]
    e = emb_ref[0]
    x = e
    for l in range(NL):
        x = _layer(x, aw_ref[l, 0], aw_ref[l, 1], aw_ref[l, 2], aw_ref[l, 3],
                   w1_ref[l], w3_ref[l], w2_ref[l], nrm_ref[l, 0], nrm_ref[l, 1],
                   cos, sin)
    out_ref[0] = e + _pool_gather(x)


def _dec_body(y_ref, aw_ref, w1_ref, w3_ref, w2_ref, nrm_ref, fn_ref, cos_ref,
              sin_ref, out_ref):
    cos = cos_ref[...]
    sin = sin_ref[...]
    x = y_ref[0]
    for l in range(NL):
        x = _layer(x, aw_ref[l, 0], aw_ref[l, 1], aw_ref[l, 2], aw_ref[l, 3],
                   w1_ref[l], w3_ref[l], w2_ref[l], nrm_ref[l, 0], nrm_ref[l, 1],
                   cos, sin)
    out_ref[0] = _rms(x, fn_ref[0])


def _full_spec(shape):
    n = len(shape)
    return pl.BlockSpec(shape, lambda b: (0,) * n)


def _tc_call(body, n_extra_specs, extra_args, x, aw, w1, w3, w2, nrm, cos, sin):
    in_specs = [
        pl.BlockSpec((1, S, D), lambda b: (b, 0, 0)),
        _full_spec(aw.shape),
        _full_spec(w1.shape),
        _full_spec(w3.shape),
        _full_spec(w2.shape),
        _full_spec(nrm.shape),
    ] + n_extra_specs + [
        _full_spec(cos.shape),
        _full_spec(sin.shape),
    ]
    return pl.pallas_call(
        body,
        grid=(B,),
        in_specs=in_specs,
        out_specs=pl.BlockSpec((1, S, D), lambda b: (b, 0, 0)),
        out_shape=jax.ShapeDtypeStruct((B, S, D), jnp.float32),
        compiler_params=pltpu.CompilerParams(
            dimension_semantics=("arbitrary",),
            vmem_limit_bytes=100 * 1024 * 1024,
        ),
    )(x, aw, w1, w3, w2, nrm, *extra_args, cos, sin)


def kernel(input_ids, tok_emb, hash_emb, enc_attn_w, enc_w1, enc_w3, enc_w2,
           enc_norms, dec_attn_w, dec_w1, dec_w3, dec_w2, dec_norms, final_norm):
    ids_flat = input_ids.astype(jnp.int32).reshape(B * S)

    embeds = _sc_embeds(ids_flat, tok_emb, hash_emb).reshape(B, S, D)

    pos = jnp.arange(S, dtype=jnp.float32)
    freqs = 1.0 / (10000.0 ** (jnp.arange(D // 2, dtype=jnp.float32) / (D // 2)))
    ang = pos[:, None] * freqs[None, :]
    cos, sin = jnp.cos(ang), jnp.sin(ang)

    bf = jnp.bfloat16
    y0 = _tc_call(_enc_body, [], [], embeds, enc_attn_w.astype(bf),
                  enc_w1.astype(bf), enc_w3.astype(bf), enc_w2.astype(bf),
                  enc_norms, cos, sin)

    fn2 = final_norm.reshape(1, D)
    out = _tc_call(_dec_body, [_full_spec(fn2.shape)], [fn2], y0,
                   dec_attn_w.astype(bf), dec_w1.astype(bf), dec_w3.astype(bf),
                   dec_w2.astype(bf), dec_norms, cos, sin)
    return out


# speedup vs baseline: 1.5205x; 1.0103x over previous
import functools
import math

import jax
import jax.numpy as jnp
from jax import lax
from jax.experimental import pallas as pl
from jax.experimental.pallas import tpu as pltpu
from jax.experimental.pallas import tpu_sc as plsc

B, S, D = 16, 512, 512
HV = 50002
NF = 3
GROUP = 4
FF = 1376
NL = 2
PATCH = 6
PRIMES = (31, 53, 97)
MULTS = tuple(tuple(pow(p, i, HV) for i in range(GROUP)) for p in PRIMES)

_R = 2
_NC, _NS = 2, 16
_NW = _NC * _NS
_TOK_PER_W = (B * S) // _NW
_CHUNK = 64


def _sc_embeds_body(ids_hbm, tok_hbm, hash_hbm, out_hbm, ids_v, hidx_v, acc_v,
                    buf_v, sem):
    cid = lax.axis_index("c")
    sid = lax.axis_index("s")
    wid = cid * _NS + sid
    base = wid * _TOK_PER_W

    ids_v[pl.ds(0, 16)] = jnp.zeros((16,), jnp.int32)
    pltpu.sync_copy(ids_hbm.at[pl.ds(base, _TOK_PER_W)], ids_v.at[pl.ds(8, _TOK_PER_W)])

    @pl.when(wid % 2 == 1)
    def _():
        pltpu.sync_copy(ids_hbm.at[pl.ds(base - 8, 8)], ids_v.at[pl.ds(0, 8)])

    for c in range(_TOK_PER_W // _CHUNK):
        for v in range(_CHUNK // 16):
            t0 = 8 + c * _CHUNK + v * 16
            w0 = ids_v[pl.ds(t0, 16)]
            w1 = ids_v[pl.ds(t0 - 1, 16)]
            w2 = ids_v[pl.ds(t0 - 2, 16)]
            w3 = ids_v[pl.ds(t0 - 3, 16)]
            for f in range(NF):
                m = MULTS[f]
                h = (w0 * m[0] + w1 * m[1] + w2 * m[2] + w3 * m[3]) % HV
                hidx_v[pl.ds(f * _CHUNK + v * 16, 16)] = h
        pltpu.async_copy(tok_hbm.at[ids_v.at[pl.ds(8 + c * _CHUNK, _CHUNK)]],
                         acc_v, sem).wait()
        for f in range(NF):
            pltpu.async_copy(hash_hbm.at[f].at[hidx_v.at[pl.ds(f * _CHUNK, _CHUNK)]],
                             buf_v, sem).wait()

            @plsc.parallel_loop(0, _CHUNK)
            def _(r):
                for j in range(D // 16):
                    plsc.addupdate(acc_v.at[r, pl.ds(j * 16, 16)],
                                   buf_v[r, pl.ds(j * 16, 16)])
        pltpu.sync_copy(acc_v, out_hbm.at[pl.ds(base + c * _CHUNK, _CHUNK)])


def _sc_embeds(ids_flat, tok_emb, hash_flat):
    mesh = plsc.VectorSubcoreMesh(core_axis_name="c", subcore_axis_name="s")
    fn = functools.partial(
        pl.kernel,
        out_type=jax.ShapeDtypeStruct((B * S, D), jnp.float32),
        mesh=mesh,
        scratch_types=[
            pltpu.VMEM((264,), jnp.int32),
            pltpu.VMEM((NF * _CHUNK,), jnp.int32),
            pltpu.VMEM((_CHUNK, D), jnp.float32),
            pltpu.VMEM((_CHUNK, D), jnp.float32),
            pltpu.SemaphoreType.DMA,
        ],
    )(_sc_embeds_body)
    return fn(ids_flat, tok_emb, hash_flat)


def _rms(x, g):
    return x * lax.rsqrt(jnp.mean(x * x, axis=-1, keepdims=True) + 1e-5) * g


def _rope(x, cos, sin):
    x1 = x[:, : D // 2]
    x2 = x[:, D // 2:]
    return jnp.concatenate([x1 * cos - x2 * sin, x1 * sin + x2 * cos], axis=-1)


def _fdot(a16, b16):
    return jnp.dot(a16, b16, preferred_element_type=jnp.float32)


def _attend(q16, k16, v16):
    s = lax.dot_general(q16, k16, (((1,), (1,)), ((), ())),
                        preferred_element_type=jnp.float32)
    s = s * (1.0 / math.sqrt(float(D)))
    rows = lax.broadcasted_iota(jnp.int32, (S, S), 0)
    cols = lax.broadcasted_iota(jnp.int32, (S, S), 1)
    s = jnp.where(rows >= cols, s, jnp.float32(-1e9))
    s = s - jnp.max(s, axis=-1, keepdims=True)
    e = jnp.exp(s)
    attn16 = (e / jnp.sum(e, axis=-1, keepdims=True)).astype(jnp.bfloat16)
    return _fdot(attn16, v16)


def _layer(x, wq, wk, wv, wo, w1, w3, w2, g0, g1, cos, sin):
    bf = jnp.bfloat16
    h16 = _rms(x, g0).astype(bf)
    q16 = _rope(_fdot(h16, wq), cos, sin).astype(bf)
    k16 = _rope(_fdot(h16, wk), cos, sin).astype(bf)
    v16 = _fdot(h16, wv).astype(bf)
    av16 = jnp.concatenate(
        [_attend(q16[r * S:(r + 1) * S], k16[r * S:(r + 1) * S],
                 v16[r * S:(r + 1) * S]) for r in range(_R)],
        axis=0).astype(bf)
    x = x + _fdot(av16, wo)
    h16 = _rms(x, g1).astype(bf)
    ff16 = (jax.nn.silu(_fdot(h16, w1)) * _fdot(h16, w3)).astype(bf)
    return x + _fdot(ff16, w2)


def _pool_gather(x):
    t = lax.broadcasted_iota(jnp.int32, (S, 1), 0)
    g = x
    ninf = jnp.float32(-jnp.inf)
    for d in range(1, PATCH):
        up = jnp.concatenate([x[d:], x[:d]], axis=0)
        cu = (t + d < S) & ((t + d) // PATCH == t // PATCH)
        g = jnp.maximum(g, jnp.where(cu, up, ninf))
        dn = jnp.concatenate([x[S - d:], x[:S - d]], axis=0)
        cd = (t - d >= 0) & ((t - d) // PATCH == t // PATCH)
        g = jnp.maximum(g, jnp.where(cd, dn, ninf))
    return g


def _enc_body(emb_ref, aw_ref, w1_ref, w3_ref, w2_ref, nrm_ref, cos_ref, sin_ref,
              out_ref):
    cos = cos_ref[...]
    sin = sin_ref[...]
    e = emb_ref[...].reshape(_R * S, D)
    x = e
    for l in range(NL):
        x = _layer(x, aw_ref[l, 0], aw_ref[l, 1], aw_ref[l, 2], aw_ref[l, 3],
                   w1_ref[l], w3_ref[l], w2_ref[l], nrm_ref[l, 0], nrm_ref[l, 1],
                   cos, sin)
    g = jnp.concatenate(
        [_pool_gather(x[r * S:(r + 1) * S]) for r in range(_R)], axis=0)
    out_ref[...] = (e + g).reshape(_R, S, D)


def _dec_body(y_ref, aw_ref, w1_ref, w3_ref, w2_ref, nrm_ref, fn_ref, cos_ref,
              sin_ref, out_ref):
    cos = cos_ref[...]
    sin = sin_ref[...]
    x = y_ref[...].reshape(_R * S, D)
    for l in range(NL):
        x = _layer(x, aw_ref[l, 0], aw_ref[l, 1], aw_ref[l, 2], aw_ref[l, 3],
                   w1_ref[l], w3_ref[l], w2_ref[l], nrm_ref[l, 0], nrm_ref[l, 1],
                   cos, sin)
    out_ref[...] = _rms(x, fn_ref[0]).reshape(_R, S, D)


def _full_spec(shape):
    n = len(shape)
    return pl.BlockSpec(shape, lambda b: (0,) * n)


def _tc_call(body, n_extra_specs, extra_args, x, aw, w1, w3, w2, nrm, cos, sin):
    in_specs = [
        pl.BlockSpec((_R, S, D), lambda b: (b, 0, 0)),
        _full_spec(aw.shape),
        _full_spec(w1.shape),
        _full_spec(w3.shape),
        _full_spec(w2.shape),
        _full_spec(nrm.shape),
    ] + n_extra_specs + [
        _full_spec(cos.shape),
        _full_spec(sin.shape),
    ]
    return pl.pallas_call(
        body,
        grid=(B // _R,),
        in_specs=in_specs,
        out_specs=pl.BlockSpec((_R, S, D), lambda b: (b, 0, 0)),
        out_shape=jax.ShapeDtypeStruct((B, S, D), jnp.float32),
        compiler_params=pltpu.CompilerParams(
            dimension_semantics=("arbitrary",),
            vmem_limit_bytes=100 * 1024 * 1024,
        ),
    )(x, aw, w1, w3, w2, nrm, *extra_args, cos, sin)


def kernel(input_ids, tok_emb, hash_emb, enc_attn_w, enc_w1, enc_w3, enc_w2,
           enc_norms, dec_attn_w, dec_w1, dec_w3, dec_w2, dec_norms, final_norm):
    ids_flat = input_ids.astype(jnp.int32).reshape(B * S)

    embeds = _sc_embeds(ids_flat, tok_emb, hash_emb).reshape(B, S, D)

    pos = jnp.arange(S, dtype=jnp.float32)
    freqs = 1.0 / (10000.0 ** (jnp.arange(D // 2, dtype=jnp.float32) / (D // 2)))
    ang = pos[:, None] * freqs[None, :]
    cos, sin = jnp.cos(ang), jnp.sin(ang)
    cos = jnp.concatenate([cos] * _R, axis=0)
    sin = jnp.concatenate([sin] * _R, axis=0)

    bf = jnp.bfloat16
    y0 = _tc_call(_enc_body, [], [], embeds, enc_attn_w.astype(bf),
                  enc_w1.astype(bf), enc_w3.astype(bf), enc_w2.astype(bf),
                  enc_norms, cos, sin)

    fn2 = final_norm.reshape(1, D)
    out = _tc_call(_dec_body, [_full_spec(fn2.shape)], [fn2], y0,
                   dec_attn_w.astype(bf), dec_w1.astype(bf), dec_w3.astype(bf),
                   dec_w2.astype(bf), dec_norms, cos, sin)
    return out
